# Initial kernel scaffold; baseline (speedup 1.0000x reference)
import jax
import jax.numpy as jnp
from jax.experimental import pallas as pl


def kernel(z_e, embed_w):
    raise NotImplementedError("write your pallas kernel here")


# speedup vs baseline: 1.1723x; 1.1723x over previous
import jax
import jax.numpy as jnp
from jax.experimental import pallas as pl

N_CODES = 1024
DIM = 128
HW = 1024
BATCH = 16


def _vq_body(z_ref, e_ref, q_ref, idx_ref, loss_ref):
    b = pl.program_id(0)
    z = z_ref[0]
    z = jnp.nan_to_num(z, nan=0.0, posinf=1.0, neginf=-1.0)
    e = e_ref[...]

    in_norm = jnp.sum(z * z, axis=0, keepdims=True)
    e_norm = jnp.sum(e * e, axis=1, keepdims=True)
    dot_t = jax.lax.dot_general(e, z, (((1,), (0,)), ((), ())))
    d = in_norm + e_norm - 2.0 * dot_t
    d = jnp.maximum(d, 0.0)

    d_min = jnp.min(d, axis=0, keepdims=True)
    code_iota = jax.lax.broadcasted_iota(jnp.int32, (N_CODES, HW), 0)
    masked = jnp.where(d == d_min, code_iota, N_CODES)
    idx = jnp.min(masked, axis=0, keepdims=True)

    onehot = (code_iota == idx).astype(jnp.float32)
    q_t = jax.lax.dot_general(
        e, onehot, (((0,), (0,)), ((), ())),
        precision=jax.lax.Precision.HIGHEST,
        preferred_element_type=jnp.float32,
    )

    diff = q_t - z
    q_ref[0] = z + diff
    idx_ref[0] = idx

    @pl.when(b == 0)
    def _init():
        loss_ref[...] = jnp.zeros((1, 1), jnp.float32)

    loss_ref[...] += jnp.sum(diff * diff, keepdims=True)


def kernel(z_e, embed_w):
    z3 = z_e.reshape(BATCH, DIM, HW)
    q3, idx3, loss = pl.pallas_call(
        _vq_body,
        grid=(BATCH,),
        in_specs=[
            pl.BlockSpec((1, DIM, HW), lambda b: (b, 0, 0)),
            pl.BlockSpec((N_CODES, DIM), lambda b: (0, 0)),
        ],
        out_specs=[
            pl.BlockSpec((1, DIM, HW), lambda b: (b, 0, 0)),
            pl.BlockSpec((1, 1, HW), lambda b: (b, 0, 0)),
            pl.BlockSpec((1, 1), lambda b: (0, 0)),
        ],
        out_shape=[
            jax.ShapeDtypeStruct((BATCH, DIM, HW), jnp.float32),
            jax.ShapeDtypeStruct((BATCH, 1, HW), jnp.int32),
            jax.ShapeDtypeStruct((1, 1), jnp.float32),
        ],
    )(z3, embed_w)
    quantized_st = q3.reshape(z_e.shape)
    indices = idx3.reshape(BATCH, HW)
    n_elems = BATCH * DIM * HW
    commitment = (loss[0, 0] / n_elems) * 0.25
    return (quantized_st, indices, commitment)


# speedup vs baseline: 1.5818x; 1.3493x over previous
import jax
import jax.numpy as jnp
from jax.experimental import pallas as pl

N_CODES = 1024
DIM = 128
HW = 1024
BATCH = 16


def _vq_body(z_ref, e_ref, q_ref, idx_ref, loss_ref):
    b = pl.program_id(0)
    z = z_ref[0]
    z = jnp.nan_to_num(z, nan=0.0, posinf=1.0, neginf=-1.0)
    e = e_ref[...]

    in_norm = jnp.sum(z * z, axis=0, keepdims=True)
    e_norm = jnp.sum(e * e, axis=1, keepdims=True)
    dot_t = jax.lax.dot_general(e, z, (((1,), (0,)), ((), ())))
    d = in_norm + e_norm - 2.0 * dot_t
    d = jnp.maximum(d, 0.0)

    d_min = jnp.min(d, axis=0, keepdims=True)
    code_iota = jax.lax.broadcasted_iota(jnp.int32, (N_CODES, HW), 0)
    masked = jnp.where(d == d_min, code_iota, N_CODES)
    idx = jnp.min(masked, axis=0, keepdims=True)

    onehot = (code_iota == idx).astype(jnp.bfloat16)
    e_hi = e.astype(jnp.bfloat16)
    e_lo = (e - e_hi.astype(jnp.float32)).astype(jnp.bfloat16)
    dims = (((0,), (0,)), ((), ()))
    q_t = (
        jax.lax.dot_general(e_hi, onehot, dims, preferred_element_type=jnp.float32)
        + jax.lax.dot_general(e_lo, onehot, dims, preferred_element_type=jnp.float32)
    )

    diff = q_t - z
    q_ref[0] = z + diff
    idx_ref[0] = idx

    @pl.when(b == 0)
    def _init():
        loss_ref[...] = jnp.zeros((1, 1), jnp.float32)

    loss_ref[...] += jnp.sum(diff * diff, keepdims=True)


def kernel(z_e, embed_w):
    z3 = z_e.reshape(BATCH, DIM, HW)
    q3, idx3, loss = pl.pallas_call(
        _vq_body,
        grid=(BATCH,),
        in_specs=[
            pl.BlockSpec((1, DIM, HW), lambda b: (b, 0, 0)),
            pl.BlockSpec((N_CODES, DIM), lambda b: (0, 0)),
        ],
        out_specs=[
            pl.BlockSpec((1, DIM, HW), lambda b: (b, 0, 0)),
            pl.BlockSpec((1, 1, HW), lambda b: (b, 0, 0)),
            pl.BlockSpec((1, 1), lambda b: (0, 0)),
        ],
        out_shape=[
            jax.ShapeDtypeStruct((BATCH, DIM, HW), jnp.float32),
            jax.ShapeDtypeStruct((BATCH, 1, HW), jnp.int32),
            jax.ShapeDtypeStruct((1, 1), jnp.float32),
        ],
    )(z3, embed_w)
    quantized_st = q3.reshape(z_e.shape)
    indices = idx3.reshape(BATCH, HW)
    n_elems = BATCH * DIM * HW
    commitment = (loss[0, 0] / n_elems) * 0.25
    return (quantized_st, indices, commitment)


# speedup vs baseline: 1.6824x; 1.0636x over previous
import jax
import jax.numpy as jnp
from jax.experimental import pallas as pl

N_CODES = 1024
DIM = 128
HW = 1024
BATCH = 16


def _vq_body(z_ref, e_ref, iota_ref, q_ref, idx_ref, loss_ref):
    b = pl.program_id(0)
    z = z_ref[0]
    e = e_ref[...]

    in_norm = jnp.sum(z * z, axis=0, keepdims=True)
    e_norm = jnp.sum(e * e, axis=1, keepdims=True)
    e2 = e + e
    dot2_t = jax.lax.dot_general(e2, z, (((1,), (0,)), ((), ())))
    d = (in_norm + e_norm) - dot2_t

    d_min = jnp.min(d, axis=0, keepdims=True)
    code_iota = iota_ref[...]
    masked = jnp.where(d == d_min, code_iota, float(N_CODES))
    idx_f = jnp.min(masked, axis=0, keepdims=True)
    idx = idx_f.astype(jnp.int32)

    onehot = (code_iota == idx_f).astype(jnp.bfloat16)
    e_hi = e.astype(jnp.bfloat16)
    e_lo = (e - e_hi.astype(jnp.float32)).astype(jnp.bfloat16)
    dims = (((0,), (0,)), ((), ()))
    q_t = (
        jax.lax.dot_general(e_hi, onehot, dims, preferred_element_type=jnp.float32)
        + jax.lax.dot_general(e_lo, onehot, dims, preferred_element_type=jnp.float32)
    )

    diff = q_t - z
    q_ref[0] = z + diff
    idx_ref[0] = idx

    @pl.when(b == 0)
    def _init():
        loss_ref[...] = jnp.zeros((1, 1), jnp.float32)

    loss_ref[...] += jnp.sum(diff * diff, keepdims=True)


def kernel(z_e, embed_w):
    z3 = z_e.reshape(BATCH, DIM, HW)
    iota_col = jnp.arange(N_CODES, dtype=jnp.float32).reshape(N_CODES, 1)
    q3, idx3, loss = pl.pallas_call(
        _vq_body,
        grid=(BATCH,),
        in_specs=[
            pl.BlockSpec((1, DIM, HW), lambda b: (b, 0, 0)),
            pl.BlockSpec((N_CODES, DIM), lambda b: (0, 0)),
            pl.BlockSpec((N_CODES, 1), lambda b: (0, 0)),
        ],
        out_specs=[
            pl.BlockSpec((1, DIM, HW), lambda b: (b, 0, 0)),
            pl.BlockSpec((1, 1, HW), lambda b: (b, 0, 0)),
            pl.BlockSpec((1, 1), lambda b: (0, 0)),
        ],
        out_shape=[
            jax.ShapeDtypeStruct((BATCH, DIM, HW), jnp.float32),
            jax.ShapeDtypeStruct((BATCH, 1, HW), jnp.int32),
            jax.ShapeDtypeStruct((1, 1), jnp.float32),
        ],
    )(z3, embed_w, iota_col)
    quantized_st = q3.reshape(z_e.shape)
    indices = idx3.reshape(BATCH, HW)
    n_elems = BATCH * DIM * HW
    commitment = (loss[0, 0] / n_elems) * 0.25
    return (quantized_st, indices, commitment)
